# baseline (speedup 1.0000x reference)
import functools

import jax
import jax.numpy as jnp
from jax import lax
from jax.experimental import pallas as pl
from jax.experimental.pallas import tpu as pltpu
from jax.experimental.pallas import tpu_sc as plsc

N = 10000
D = 128
E = 320000

NC = 2
NS = 16
NW = NC * NS

CHUNK = 128
KCH0 = 56
KCH1 = 104
KCHMAX = 104
TOT_CH = NS * (KCH0 + KCH1)
E_PAD = TOT_CH * CHUNK
KCH_D = E_PAD // (NW * CHUNK)
N_PAD = 10240
RPT = N_PAD // NS
ROWBLK = 512
GRID = N_PAD // ROWBLK

_mesh = plsc.VectorSubcoreMesh(core_axis_name="c", subcore_axis_name="s")


def _zero_fill(ref, nrow):
    z = jnp.zeros((16,), jnp.float32)

    def body(i, _):
        r = i // 8
        c = i % 8
        ref[r, pl.ds(c * 16, 16)] = z
        return 0

    lax.fori_loop(0, nrow * 8, body, 0)


@functools.partial(
    pl.kernel,
    mesh=_mesh,
    out_type=jax.ShapeDtypeStruct((NC * N_PAD,), jnp.float32),
    scratch_types=[
        pltpu.VMEM((KCH_D, CHUNK), jnp.int32),
        pltpu.VMEM((KCH_D, CHUNK), jnp.float32),
        pltpu.VMEM((RPT,), jnp.float32),
        pltpu.VMEM_SHARED((N_PAD,), jnp.float32),
    ],
)
def _sc_degree(dst_hbm, val_hbm, out_hbm, dstv, valv, zerov, acc):
    cid = lax.axis_index("c")
    sid = lax.axis_index("s")
    wid = cid * NS + sid

    z = jnp.zeros((16,), jnp.float32)

    def zbody(i, _):
        zerov[pl.ds(i * 16, 16)] = z
        return 0

    lax.fori_loop(0, RPT // 16, zbody, 0)
    pltpu.sync_copy(zerov, acc.at[pl.ds(sid * RPT, RPT)])
    plsc.subcore_barrier()

    pltpu.sync_copy(dst_hbm.at[wid], dstv)
    pltpu.sync_copy(val_hbm.at[wid], valv)

    def body(j, _):
        pltpu.sync_copy(valv.at[j], acc.at[dstv.at[j]], add=True)
        return 0

    lax.fori_loop(0, KCH_D, body, 0)
    plsc.subcore_barrier()

    pltpu.sync_copy(acc.at[pl.ds(sid * RPT, RPT)],
                    out_hbm.at[pl.ds(cid * N_PAD + sid * RPT, RPT)])


@functools.partial(
    pl.kernel,
    mesh=_mesh,
    out_type=jax.ShapeDtypeStruct((NC * N_PAD, D), jnp.float32),
    scratch_types=[
        pltpu.VMEM((KCHMAX, CHUNK), jnp.int32),
        pltpu.VMEM((KCHMAX, CHUNK), jnp.int32),
        pltpu.VMEM((CHUNK, D), jnp.float32),
        pltpu.VMEM_SHARED((N_PAD, D), jnp.float32),
        pltpu.SemaphoreType.DMA,
    ],
)
def _sc_prop(hs_hbm, src_hbm, dst_hbm, out_hbm, srcv, dstv, rows, acc, sem):
    cid = lax.axis_index("c")
    sid = lax.axis_index("s")

    _zero_fill(rows, CHUNK)

    def zcopy(k, _):
        pltpu.sync_copy(rows, acc.at[pl.ds(sid * RPT + k * CHUNK, CHUNK)])
        return 0

    lax.fori_loop(0, RPT // CHUNK, zcopy, 0)
    plsc.subcore_barrier()

    base = jnp.where(cid == 0, sid * KCH0, NS * KCH0 + sid * KCH1)
    base = pl.multiple_of(base, 8)
    nch = jnp.where(cid == 0, KCH0, KCH1)
    pltpu.sync_copy(src_hbm.at[pl.ds(base, KCHMAX)], srcv)
    pltpu.sync_copy(dst_hbm.at[pl.ds(base, KCHMAX)], dstv)

    def body(j, _):
        pltpu.async_copy(hs_hbm.at[srcv.at[j]], rows, sem).wait()
        pltpu.sync_copy(rows, acc.at[dstv.at[j]], add=True)
        return 0

    lax.fori_loop(0, nch, body, 0)
    plsc.subcore_barrier()

    pltpu.sync_copy(acc.at[pl.ds(sid * RPT, RPT)],
                    out_hbm.at[pl.ds(cid * N_PAD + sid * RPT, RPT)])


def _tc_prep_body(x_ref, d0_ref, d1_ref, dinv_ref, hs_ref):
    i = pl.program_id(0)
    row = lax.broadcasted_iota(jnp.int32, (ROWBLK, 1), 0) + i * ROWBLK
    real = (row < N).astype(jnp.float32)
    deg = d0_ref[...] + d1_ref[...] + real
    dinv = jnp.where(deg > 0, lax.rsqrt(deg), 0.0)
    dinv_b = jnp.broadcast_to(dinv, (ROWBLK, D))
    dinv_ref[...] = dinv_b
    hs_ref[...] = dinv_b * x_ref[...]


def _tc_layer_body(a0_ref, a1_ref, hs_ref, dinv_ref, w_ref, b_ref, *rest,
                   relu, has_mprev, emit_hs):
    if has_mprev:
        m_ref_in = rest[0]
        outs = rest[1:]
    else:
        m_ref_in = None
        outs = rest
    m_out = outs[0]
    dinv = dinv_ref[...]
    g = dinv * (a0_ref[...] + a1_ref[...] + hs_ref[...])
    h = lax.dot_general(g, w_ref[...], (((1,), (1,)), ((), ())),
                        preferred_element_type=jnp.float32) + b_ref[...]
    if relu:
        h = jnp.maximum(h, 0.0)
    m = jnp.maximum(m_ref_in[...], h) if has_mprev else h
    m_out[...] = m
    if emit_hs:
        outs[1][...] = dinv * h


_blk = pl.BlockSpec((ROWBLK, D), lambda i: (i, 0))
_blk1 = pl.BlockSpec((ROWBLK, 1), lambda i: (i, 0))
_blkW = pl.BlockSpec((D, D), lambda i: (0, 0))
_blkb = pl.BlockSpec((1, D), lambda i: (0, 0))
_shape = jax.ShapeDtypeStruct((N_PAD, D), jnp.float32)


def _tc_prep(x_pad, deg0, deg1):
    return pl.pallas_call(
        _tc_prep_body,
        grid=(GRID,),
        in_specs=[_blk, _blk1, _blk1],
        out_specs=[_blk, _blk],
        out_shape=[_shape, _shape],
    )(x_pad, deg0, deg1)


def _tc_layer(a0, a1, hs, dinv_b, w, b, m_prev, relu, emit_hs):
    has_mprev = m_prev is not None
    body = functools.partial(_tc_layer_body, relu=relu,
                             has_mprev=has_mprev, emit_hs=emit_hs)
    in_specs = [_blk, _blk, _blk, _blk, _blkW, _blkb]
    args = [a0, a1, hs, dinv_b, w, b]
    if has_mprev:
        in_specs.append(_blk)
        args.append(m_prev)
    out_specs = [_blk, _blk] if emit_hs else [_blk]
    out_shape = [_shape, _shape] if emit_hs else [_shape]
    return pl.pallas_call(
        body,
        grid=(GRID,),
        in_specs=in_specs,
        out_specs=out_specs,
        out_shape=out_shape,
    )(*args)


def kernel(x, adj_t, W1, b1, W2, b2, W3, b3):
    src = adj_t[0]
    dst = adj_t[1]

    x_pad = jnp.concatenate([x, jnp.zeros((N_PAD - N, D), jnp.float32)])
    src_p = jnp.concatenate(
        [src, jnp.full((E_PAD - E,), N, jnp.int32)]).reshape(TOT_CH, CHUNK)
    dst_p = jnp.concatenate(
        [dst, jnp.zeros((E_PAD - E,), jnp.int32)]).reshape(TOT_CH, CHUNK)
    dst_d = dst_p.reshape(NW, KCH_D, CHUNK)
    vals = (jnp.arange(E_PAD, dtype=jnp.int32) < E).astype(
        jnp.float32).reshape(NW, KCH_D, CHUNK)

    deg = _sc_degree(dst_d, vals)
    deg0 = deg[:N_PAD].reshape(N_PAD, 1)
    deg1 = deg[N_PAD:].reshape(N_PAD, 1)
    dinv_b, hs = _tc_prep(x_pad, deg0, deg1)

    m = None
    for (W, b, relu) in ((W1, b1, True), (W2, b2, True), (W3, b3, False)):
        a = _sc_prop(hs, src_p, dst_p)
        a0 = a[:N_PAD]
        a1 = a[N_PAD:]
        res = _tc_layer(a0, a1, hs, dinv_b, W, b.reshape(1, D),
                        m, relu, emit_hs=relu)
        if relu:
            m, hs = res[0], res[1]
        else:
            m = res[0]
    return m[:N]


# speedup vs baseline: 1.1475x; 1.1475x over previous
import functools

import jax
import jax.numpy as jnp
from jax import lax
from jax.experimental import pallas as pl
from jax.experimental.pallas import tpu as pltpu
from jax.experimental.pallas import tpu_sc as plsc

N = 10000
D = 128
E = 320000

NC = 2
NS = 16
NW = NC * NS

CHUNK = 128
KCH0 = 104
KCH1 = 56
KCHMAX = 104
TOT_CH = NS * (KCH0 + KCH1)
E_PAD = TOT_CH * CHUNK
KCH_D = E_PAD // (NW * CHUNK)
N_PAD = 10240
RPT = N_PAD // NS
ROWBLK = 512
GRID = N_PAD // ROWBLK

_mesh = plsc.VectorSubcoreMesh(core_axis_name="c", subcore_axis_name="s")


def _zero_fill(ref, nrow):
    z = jnp.zeros((16,), jnp.float32)

    def body(i, _):
        r = i // 8
        c = i % 8
        ref[r, pl.ds(c * 16, 16)] = z
        return 0

    lax.fori_loop(0, nrow * 8, body, 0)


@functools.partial(
    pl.kernel,
    mesh=_mesh,
    out_type=jax.ShapeDtypeStruct((NC * N_PAD,), jnp.float32),
    scratch_types=[
        pltpu.VMEM((KCH_D, CHUNK), jnp.int32),
        pltpu.VMEM((KCH_D, CHUNK), jnp.float32),
        pltpu.VMEM((RPT,), jnp.float32),
        pltpu.VMEM_SHARED((N_PAD,), jnp.float32),
    ],
)
def _sc_degree(dst_hbm, val_hbm, out_hbm, dstv, valv, zerov, acc):
    cid = lax.axis_index("c")
    sid = lax.axis_index("s")
    wid = cid * NS + sid

    z = jnp.zeros((16,), jnp.float32)

    def zbody(i, _):
        zerov[pl.ds(i * 16, 16)] = z
        return 0

    lax.fori_loop(0, RPT // 16, zbody, 0)
    pltpu.sync_copy(zerov, acc.at[pl.ds(sid * RPT, RPT)])
    plsc.subcore_barrier()

    pltpu.sync_copy(dst_hbm.at[wid], dstv)
    pltpu.sync_copy(val_hbm.at[wid], valv)

    def body(j, _):
        pltpu.sync_copy(valv.at[j], acc.at[dstv.at[j]], add=True)
        return 0

    lax.fori_loop(0, KCH_D, body, 0)
    plsc.subcore_barrier()

    pltpu.sync_copy(acc.at[pl.ds(sid * RPT, RPT)],
                    out_hbm.at[pl.ds(cid * N_PAD + sid * RPT, RPT)])


@functools.partial(
    pl.kernel,
    mesh=_mesh,
    out_type=jax.ShapeDtypeStruct((NC * N_PAD, D), jnp.float32),
    scratch_types=[
        pltpu.VMEM((KCHMAX, CHUNK), jnp.int32),
        pltpu.VMEM((KCHMAX, CHUNK), jnp.int32),
        pltpu.VMEM((CHUNK, D), jnp.float32),
        pltpu.VMEM_SHARED((N_PAD, D), jnp.float32),
        pltpu.SemaphoreType.DMA,
    ],
)
def _sc_prop(hs_hbm, src_hbm, dst_hbm, out_hbm, srcv, dstv, rows, acc, sem):
    cid = lax.axis_index("c")
    sid = lax.axis_index("s")

    _zero_fill(rows, CHUNK)

    def zcopy(k, _):
        pltpu.sync_copy(rows, acc.at[pl.ds(sid * RPT + k * CHUNK, CHUNK)])
        return 0

    lax.fori_loop(0, RPT // CHUNK, zcopy, 0)
    plsc.subcore_barrier()

    base = jnp.where(cid == 0, sid * KCH0, NS * KCH0 + sid * KCH1)
    base = pl.multiple_of(base, 8)
    nch = jnp.where(cid == 0, KCH0, KCH1)
    pltpu.sync_copy(src_hbm.at[pl.ds(base, KCHMAX)], srcv)
    pltpu.sync_copy(dst_hbm.at[pl.ds(base, KCHMAX)], dstv)

    def body(j, _):
        pltpu.async_copy(hs_hbm.at[srcv.at[j]], rows, sem).wait()
        pltpu.sync_copy(rows, acc.at[dstv.at[j]], add=True)
        return 0

    lax.fori_loop(0, nch, body, 0)
    plsc.subcore_barrier()

    pltpu.sync_copy(acc.at[pl.ds(sid * RPT, RPT)],
                    out_hbm.at[pl.ds(cid * N_PAD + sid * RPT, RPT)])


def _tc_prep_body(x_ref, d0_ref, d1_ref, dinv_ref, hs_ref):
    i = pl.program_id(0)
    row = lax.broadcasted_iota(jnp.int32, (ROWBLK, 1), 0) + i * ROWBLK
    real = (row < N).astype(jnp.float32)
    deg = d0_ref[...] + d1_ref[...] + real
    dinv = jnp.where(deg > 0, lax.rsqrt(deg), 0.0)
    dinv_b = jnp.broadcast_to(dinv, (ROWBLK, D))
    dinv_ref[...] = dinv_b
    hs_ref[...] = dinv_b * x_ref[...]


def _tc_layer_body(a0_ref, a1_ref, hs_ref, dinv_ref, w_ref, b_ref, *rest,
                   relu, has_mprev, emit_hs):
    if has_mprev:
        m_ref_in = rest[0]
        outs = rest[1:]
    else:
        m_ref_in = None
        outs = rest
    m_out = outs[0]
    dinv = dinv_ref[...]
    g = dinv * (a0_ref[...] + a1_ref[...] + hs_ref[...])
    h = lax.dot_general(g, w_ref[...], (((1,), (1,)), ((), ())),
                        preferred_element_type=jnp.float32) + b_ref[...]
    if relu:
        h = jnp.maximum(h, 0.0)
    m = jnp.maximum(m_ref_in[...], h) if has_mprev else h
    m_out[...] = m
    if emit_hs:
        outs[1][...] = dinv * h


_blk = pl.BlockSpec((ROWBLK, D), lambda i: (i, 0))
_blk1 = pl.BlockSpec((ROWBLK, 1), lambda i: (i, 0))
_blkW = pl.BlockSpec((D, D), lambda i: (0, 0))
_blkb = pl.BlockSpec((1, D), lambda i: (0, 0))
_shape = jax.ShapeDtypeStruct((N_PAD, D), jnp.float32)


def _tc_prep(x_pad, deg0, deg1):
    return pl.pallas_call(
        _tc_prep_body,
        grid=(GRID,),
        in_specs=[_blk, _blk1, _blk1],
        out_specs=[_blk, _blk],
        out_shape=[_shape, _shape],
    )(x_pad, deg0, deg1)


def _tc_layer(a0, a1, hs, dinv_b, w, b, m_prev, relu, emit_hs):
    has_mprev = m_prev is not None
    body = functools.partial(_tc_layer_body, relu=relu,
                             has_mprev=has_mprev, emit_hs=emit_hs)
    in_specs = [_blk, _blk, _blk, _blk, _blkW, _blkb]
    args = [a0, a1, hs, dinv_b, w, b]
    if has_mprev:
        in_specs.append(_blk)
        args.append(m_prev)
    out_specs = [_blk, _blk] if emit_hs else [_blk]
    out_shape = [_shape, _shape] if emit_hs else [_shape]
    return pl.pallas_call(
        body,
        grid=(GRID,),
        in_specs=in_specs,
        out_specs=out_specs,
        out_shape=out_shape,
    )(*args)


def kernel(x, adj_t, W1, b1, W2, b2, W3, b3):
    src = adj_t[0]
    dst = adj_t[1]

    x_pad = jnp.concatenate([x, jnp.zeros((N_PAD - N, D), jnp.float32)])
    src_p = jnp.concatenate(
        [src, jnp.full((E_PAD - E,), N, jnp.int32)]).reshape(TOT_CH, CHUNK)
    dst_p = jnp.concatenate(
        [dst, jnp.zeros((E_PAD - E,), jnp.int32)]).reshape(TOT_CH, CHUNK)
    dst_d = dst_p.reshape(NW, KCH_D, CHUNK)
    vals = (jnp.arange(E_PAD, dtype=jnp.int32) < E).astype(
        jnp.float32).reshape(NW, KCH_D, CHUNK)

    deg = _sc_degree(dst_d, vals)
    deg0 = deg[:N_PAD].reshape(N_PAD, 1)
    deg1 = deg[N_PAD:].reshape(N_PAD, 1)
    dinv_b, hs = _tc_prep(x_pad, deg0, deg1)

    m = None
    for (W, b, relu) in ((W1, b1, True), (W2, b2, True), (W3, b3, False)):
        a = _sc_prop(hs, src_p, dst_p)
        a0 = a[:N_PAD]
        a1 = a[N_PAD:]
        res = _tc_layer(a0, a1, hs, dinv_b, W, b.reshape(1, D),
                        m, relu, emit_hs=relu)
        if relu:
            m, hs = res[0], res[1]
        else:
            m = res[0]
    return m[:N]


# speedup vs baseline: 1.5864x; 1.3825x over previous
import functools

import jax
import jax.numpy as jnp
from jax import lax
from jax.experimental import pallas as pl
from jax.experimental.pallas import tpu as pltpu
from jax.experimental.pallas import tpu_sc as plsc

N = 10000
D = 128
E = 320000

NC = 2
NS = 16
NW = NC * NS

CHUNK = 128
KCH = 79
EPT = KCH * CHUNK
E_PAD = NW * EPT
N_PAD = 10240
RPT = N_PAD // NS
ROWBLK = 512
GRID = N_PAD // ROWBLK

_mesh = plsc.VectorSubcoreMesh(core_axis_name="c", subcore_axis_name="s")


def _zero_fill(ref, nrow):
    z = jnp.zeros((16,), jnp.float32)

    def body(i, _):
        r = i // 8
        c = i % 8
        ref[r, pl.ds(c * 16, 16)] = z
        return 0

    lax.fori_loop(0, nrow * 8, body, 0)


@functools.partial(
    pl.kernel,
    mesh=_mesh,
    out_type=jax.ShapeDtypeStruct((NC * N_PAD,), jnp.float32),
    scratch_types=[
        pltpu.VMEM((KCH, CHUNK), jnp.int32),
        pltpu.VMEM((KCH, CHUNK), jnp.float32),
        pltpu.VMEM((RPT,), jnp.float32),
        pltpu.VMEM_SHARED((N_PAD,), jnp.float32),
    ],
)
def _sc_degree(dst_hbm, val_hbm, out_hbm, dstv, valv, zerov, acc):
    cid = lax.axis_index("c")
    sid = lax.axis_index("s")
    wid = cid * NS + sid

    z = jnp.zeros((16,), jnp.float32)

    def zbody(i, _):
        zerov[pl.ds(i * 16, 16)] = z
        return 0

    lax.fori_loop(0, RPT // 16, zbody, 0)
    pltpu.sync_copy(zerov, acc.at[pl.ds(sid * RPT, RPT)])
    plsc.subcore_barrier()

    pltpu.sync_copy(dst_hbm.at[wid], dstv)
    pltpu.sync_copy(val_hbm.at[wid], valv)

    def body(j, _):
        pltpu.sync_copy(valv.at[j], acc.at[dstv.at[j]], add=True)
        return 0

    lax.fori_loop(0, KCH, body, 0)
    plsc.subcore_barrier()

    pltpu.sync_copy(acc.at[pl.ds(sid * RPT, RPT)],
                    out_hbm.at[pl.ds(cid * N_PAD + sid * RPT, RPT)])


@functools.partial(
    pl.kernel,
    mesh=_mesh,
    out_type=jax.ShapeDtypeStruct((NC * N_PAD, D), jnp.float32),
    scratch_types=[
        pltpu.VMEM((KCH, CHUNK), jnp.int32),
        pltpu.VMEM((KCH, CHUNK), jnp.int32),
        pltpu.VMEM((CHUNK, D), jnp.float32),
        pltpu.VMEM_SHARED((N_PAD, D), jnp.float32),
        pltpu.SemaphoreType.DMA,
    ],
)
def _sc_prop(hs_hbm, src_hbm, dst_hbm, out_hbm, srcv, dstv, rows, acc, sem):
    cid = lax.axis_index("c")
    sid = lax.axis_index("s")
    wid = cid * NS + sid

    _zero_fill(rows, CHUNK)

    def zcopy(k, _):
        pltpu.sync_copy(rows, acc.at[pl.ds(sid * RPT + k * CHUNK, CHUNK)])
        return 0

    lax.fori_loop(0, RPT // CHUNK, zcopy, 0)
    plsc.subcore_barrier()

    pltpu.sync_copy(src_hbm.at[wid], srcv)
    pltpu.sync_copy(dst_hbm.at[wid], dstv)

    def body(j, _):
        pltpu.async_copy(hs_hbm.at[srcv.at[j]], rows, sem).wait()
        pltpu.sync_copy(rows, acc.at[dstv.at[j]], add=True)
        return 0

    lax.fori_loop(0, KCH, body, 0)
    plsc.subcore_barrier()

    pltpu.sync_copy(acc.at[pl.ds(sid * RPT, RPT)],
                    out_hbm.at[pl.ds(cid * N_PAD + sid * RPT, RPT)])


def _tc_prep_body(x_ref, d0_ref, d1_ref, dinv_ref, hs_ref):
    i = pl.program_id(0)
    row = lax.broadcasted_iota(jnp.int32, (ROWBLK, 1), 0) + i * ROWBLK
    real = (row < N).astype(jnp.float32)
    deg = d0_ref[...] + d1_ref[...] + real
    dinv = jnp.where(deg > 0, lax.rsqrt(deg), 0.0)
    dinv_ref[...] = dinv
    hs_ref[...] = dinv * x_ref[...]


def _tc_layer_body(a0_ref, a1_ref, hs_ref, dinv_ref, w_ref, b_ref, *rest,
                   relu, has_mprev, emit_hs):
    if has_mprev:
        m_ref_in = rest[0]
        outs = rest[1:]
    else:
        m_ref_in = None
        outs = rest
    m_out = outs[0]
    dinv = dinv_ref[...]
    g = dinv * (a0_ref[...] + a1_ref[...] + hs_ref[...])
    h = lax.dot_general(g, w_ref[...], (((1,), (1,)), ((), ())),
                        preferred_element_type=jnp.float32) + b_ref[...]
    if relu:
        h = jnp.maximum(h, 0.0)
    m = jnp.maximum(m_ref_in[...], h) if has_mprev else h
    m_out[...] = m
    if emit_hs:
        outs[1][...] = dinv * h


_blk = pl.BlockSpec((ROWBLK, D), lambda i: (i, 0))
_blk1 = pl.BlockSpec((ROWBLK, 1), lambda i: (i, 0))
_blkW = pl.BlockSpec((D, D), lambda i: (0, 0))
_blkb = pl.BlockSpec((1, D), lambda i: (0, 0))
_shape = jax.ShapeDtypeStruct((N_PAD, D), jnp.float32)


def _tc_prep(x_pad, deg0, deg1):
    return pl.pallas_call(
        _tc_prep_body,
        grid=(GRID,),
        in_specs=[_blk, _blk1, _blk1],
        out_specs=[_blk1, _blk],
        out_shape=[jax.ShapeDtypeStruct((N_PAD, 1), jnp.float32), _shape],
    )(x_pad, deg0, deg1)


def _tc_layer(a0, a1, hs, dinv_b, w, b, m_prev, relu, emit_hs):
    has_mprev = m_prev is not None
    body = functools.partial(_tc_layer_body, relu=relu,
                             has_mprev=has_mprev, emit_hs=emit_hs)
    in_specs = [_blk, _blk, _blk, _blk1, _blkW, _blkb]
    args = [a0, a1, hs, dinv_b, w, b]
    if has_mprev:
        in_specs.append(_blk)
        args.append(m_prev)
    out_specs = [_blk, _blk] if emit_hs else [_blk]
    out_shape = [_shape, _shape] if emit_hs else [_shape]
    return pl.pallas_call(
        body,
        grid=(GRID,),
        in_specs=in_specs,
        out_specs=out_specs,
        out_shape=out_shape,
    )(*args)


def kernel(x, adj_t, W1, b1, W2, b2, W3, b3):
    src = adj_t[0]
    dst = adj_t[1]

    x_pad = jnp.concatenate([x, jnp.zeros((N_PAD - N, D), jnp.float32)])
    src_p = jnp.concatenate(
        [src, jnp.full((E_PAD - E,), N, jnp.int32)]).reshape(NW, KCH, CHUNK)
    dst_p = jnp.concatenate(
        [dst, jnp.zeros((E_PAD - E,), jnp.int32)]).reshape(NW, KCH, CHUNK)
    vals = (jnp.arange(E_PAD, dtype=jnp.int32) < E).astype(
        jnp.float32).reshape(NW, KCH, CHUNK)

    deg = _sc_degree(dst_p, vals)
    deg0 = deg[:N_PAD].reshape(N_PAD, 1)
    deg1 = deg[N_PAD:].reshape(N_PAD, 1)
    dinv_b, hs = _tc_prep(x_pad, deg0, deg1)

    m = None
    for (W, b, relu) in ((W1, b1, True), (W2, b2, True), (W3, b3, False)):
        a = _sc_prop(hs, src_p, dst_p)
        a0 = a[:N_PAD]
        a1 = a[N_PAD:]
        res = _tc_layer(a0, a1, hs, dinv_b, W, b.reshape(1, D),
                        m, relu, emit_hs=relu)
        if relu:
            m, hs = res[0], res[1]
        else:
            m = res[0]
    return m[:N]


# speedup vs baseline: 1.8960x; 1.1951x over previous
import functools

import jax
import jax.numpy as jnp
from jax import lax
from jax.experimental import pallas as pl
from jax.experimental.pallas import tpu as pltpu
from jax.experimental.pallas import tpu_sc as plsc

N = 10000
D = 128
E = 320000

NC = 2
NS = 16
NW = NC * NS

CHUNK = 128
KCH = 79
KGRP = 40
EPT = KCH * CHUNK
E_PAD = NW * EPT
N_PAD = 10240
RPT = N_PAD // NS
ROWBLK = 512
GRID = N_PAD // ROWBLK

_mesh = plsc.VectorSubcoreMesh(core_axis_name="c", subcore_axis_name="s")


def _zero_fill(ref, nrow):
    z = jnp.zeros((16,), jnp.float32)

    def body(i, _):
        r = i // 8
        c = i % 8
        ref[r, pl.ds(c * 16, 16)] = z
        return 0

    lax.fori_loop(0, nrow * 8, body, 0)


@functools.partial(
    pl.kernel,
    mesh=_mesh,
    out_type=jax.ShapeDtypeStruct((NC * N_PAD,), jnp.float32),
    scratch_types=[
        pltpu.VMEM((KCH, CHUNK), jnp.int32),
        pltpu.VMEM((KCH, CHUNK), jnp.float32),
        pltpu.VMEM((RPT,), jnp.float32),
        pltpu.VMEM_SHARED((N_PAD,), jnp.float32),
    ],
)
def _sc_degree(dst_hbm, val_hbm, out_hbm, dstv, valv, zerov, acc):
    cid = lax.axis_index("c")
    sid = lax.axis_index("s")
    wid = cid * NS + sid

    z = jnp.zeros((16,), jnp.float32)

    def zbody(i, _):
        zerov[pl.ds(i * 16, 16)] = z
        return 0

    lax.fori_loop(0, RPT // 16, zbody, 0)
    pltpu.sync_copy(zerov, acc.at[pl.ds(sid * RPT, RPT)])
    plsc.subcore_barrier()

    pltpu.sync_copy(dst_hbm.at[wid], dstv)
    pltpu.sync_copy(val_hbm.at[wid], valv)

    def body(j, _):
        pltpu.sync_copy(valv.at[j], acc.at[dstv.at[j]], add=True)
        return 0

    lax.fori_loop(0, KCH, body, 0)
    plsc.subcore_barrier()

    pltpu.sync_copy(acc.at[pl.ds(sid * RPT, RPT)],
                    out_hbm.at[pl.ds(cid * N_PAD + sid * RPT, RPT)])


@functools.partial(
    pl.kernel,
    mesh=_mesh,
    out_type=jax.ShapeDtypeStruct((NC * N_PAD, D), jnp.float32),
    scratch_types=[
        pltpu.VMEM((KGRP, CHUNK), jnp.int32),
        pltpu.VMEM((KGRP, CHUNK), jnp.int32),
        pltpu.VMEM((2, CHUNK, D), jnp.float32),
        pltpu.VMEM_SHARED((N_PAD, D), jnp.float32),
        pltpu.SemaphoreType.DMA,
        pltpu.SemaphoreType.DMA,
    ],
)
def _sc_prop(hs_hbm, src_hbm, dst_hbm, out_hbm, srcv, dstv, rows, acc,
             sem0, sem1):
    cid = lax.axis_index("c")
    sid = lax.axis_index("s")
    wid = cid * NS + sid

    _zero_fill(rows.at[0], CHUNK)

    def zcopy(k, _):
        pltpu.sync_copy(rows.at[0], acc.at[pl.ds(sid * RPT + k * CHUNK, CHUNK)])
        return 0

    lax.fori_loop(0, RPT // CHUNK, zcopy, 0)
    plsc.subcore_barrier()

    sems = (sem0, sem1)
    for base in range(0, KCH, KGRP):
        cnt = min(KGRP, KCH - base)
        pltpu.sync_copy(src_hbm.at[wid, pl.ds(base, cnt)],
                        srcv.at[pl.ds(0, cnt)])
        pltpu.sync_copy(dst_hbm.at[wid, pl.ds(base, cnt)],
                        dstv.at[pl.ds(0, cnt)])
        handles = [None, None]
        handles[0] = pltpu.async_copy(hs_hbm.at[srcv.at[0]], rows.at[0],
                                      sems[0])
        for j in range(cnt):
            if j + 1 < cnt:
                handles[(j + 1) % 2] = pltpu.async_copy(
                    hs_hbm.at[srcv.at[j + 1]], rows.at[(j + 1) % 2],
                    sems[(j + 1) % 2])
            handles[j % 2].wait()
            pltpu.sync_copy(rows.at[j % 2], acc.at[dstv.at[j]], add=True)
    plsc.subcore_barrier()

    pltpu.sync_copy(acc.at[pl.ds(sid * RPT, RPT)],
                    out_hbm.at[pl.ds(cid * N_PAD + sid * RPT, RPT)])


def _tc_prep_body(x_ref, d0_ref, d1_ref, dinv_ref, hs_ref):
    i = pl.program_id(0)
    row = lax.broadcasted_iota(jnp.int32, (ROWBLK, 1), 0) + i * ROWBLK
    real = (row < N).astype(jnp.float32)
    deg = d0_ref[...] + d1_ref[...] + real
    dinv = jnp.where(deg > 0, lax.rsqrt(deg), 0.0)
    dinv_ref[...] = dinv
    hs_ref[...] = dinv * x_ref[...]


def _tc_layer_body(a0_ref, a1_ref, hs_ref, dinv_ref, w_ref, b_ref, *rest,
                   relu, has_mprev, emit_hs):
    if has_mprev:
        m_ref_in = rest[0]
        outs = rest[1:]
    else:
        m_ref_in = None
        outs = rest
    m_out = outs[0]
    dinv = dinv_ref[...]
    g = dinv * (a0_ref[...] + a1_ref[...] + hs_ref[...])
    h = lax.dot_general(g, w_ref[...], (((1,), (1,)), ((), ())),
                        preferred_element_type=jnp.float32) + b_ref[...]
    if relu:
        h = jnp.maximum(h, 0.0)
    m = jnp.maximum(m_ref_in[...], h) if has_mprev else h
    m_out[...] = m
    if emit_hs:
        outs[1][...] = dinv * h


_blk = pl.BlockSpec((ROWBLK, D), lambda i: (i, 0))
_blk1 = pl.BlockSpec((ROWBLK, 1), lambda i: (i, 0))
_blkW = pl.BlockSpec((D, D), lambda i: (0, 0))
_blkb = pl.BlockSpec((1, D), lambda i: (0, 0))
_shape = jax.ShapeDtypeStruct((N_PAD, D), jnp.float32)


def _tc_prep(x_pad, deg0, deg1):
    return pl.pallas_call(
        _tc_prep_body,
        grid=(GRID,),
        in_specs=[_blk, _blk1, _blk1],
        out_specs=[_blk1, _blk],
        out_shape=[jax.ShapeDtypeStruct((N_PAD, 1), jnp.float32), _shape],
    )(x_pad, deg0, deg1)


def _tc_layer(a0, a1, hs, dinv_b, w, b, m_prev, relu, emit_hs):
    has_mprev = m_prev is not None
    body = functools.partial(_tc_layer_body, relu=relu,
                             has_mprev=has_mprev, emit_hs=emit_hs)
    in_specs = [_blk, _blk, _blk, _blk1, _blkW, _blkb]
    args = [a0, a1, hs, dinv_b, w, b]
    if has_mprev:
        in_specs.append(_blk)
        args.append(m_prev)
    out_specs = [_blk, _blk] if emit_hs else [_blk]
    out_shape = [_shape, _shape] if emit_hs else [_shape]
    return pl.pallas_call(
        body,
        grid=(GRID,),
        in_specs=in_specs,
        out_specs=out_specs,
        out_shape=out_shape,
    )(*args)


def kernel(x, adj_t, W1, b1, W2, b2, W3, b3):
    src = adj_t[0]
    dst = adj_t[1]

    x_pad = jnp.concatenate([x, jnp.zeros((N_PAD - N, D), jnp.float32)])
    src_p = jnp.concatenate(
        [src, jnp.full((E_PAD - E,), N, jnp.int32)]).reshape(NW, KCH, CHUNK)
    dst_p = jnp.concatenate(
        [dst, jnp.zeros((E_PAD - E,), jnp.int32)]).reshape(NW, KCH, CHUNK)
    vals = (jnp.arange(E_PAD, dtype=jnp.int32) < E).astype(
        jnp.float32).reshape(NW, KCH, CHUNK)

    deg = _sc_degree(dst_p, vals)
    deg0 = deg[:N_PAD].reshape(N_PAD, 1)
    deg1 = deg[N_PAD:].reshape(N_PAD, 1)
    dinv_b, hs = _tc_prep(x_pad, deg0, deg1)

    m = None
    for (W, b, relu) in ((W1, b1, True), (W2, b2, True), (W3, b3, False)):
        a = _sc_prop(hs, src_p, dst_p)
        a0 = a[:N_PAD]
        a1 = a[N_PAD:]
        res = _tc_layer(a0, a1, hs, dinv_b, W, b.reshape(1, D),
                        m, relu, emit_hs=relu)
        if relu:
            m, hs = res[0], res[1]
        else:
            m = res[0]
    return m[:N]
